# baseline (speedup 1.0000x reference)
import functools

import jax
import jax.numpy as jnp
from jax import lax
from jax.experimental import pallas as pl
from jax.experimental.pallas import tpu as pltpu
from jax.experimental.pallas import tpu_sc as plsc

N = 50000
E = 800000
HID = 40
HP = 40
POSW = 16
MPN = 15

NC = 2
NS = 16
NW = NC * NS
CH = 128
EPW = E // NW
GCH = 1000
GJ = EPW // GCH

SW = HID // NC
SCH = 640
SGRP = E // SCH
SOPS = 78
SXTR = SGRP - NS * SOPS
ZST = N // NS


def _mesh():
    return plsc.VectorSubcoreMesh(
        core_axis_name="c", subcore_axis_name="s",
        num_cores=NC, num_subcores=NS)


_SC_PARAMS = pltpu.CompilerParams(use_tc_tiling_on_sc=False)


@functools.lru_cache(maxsize=None)
def _gather2(width):
    @functools.partial(
        pl.kernel,
        out_type=[jax.ShapeDtypeStruct((E, width), jnp.float32)] * 2,
        scratch_types=(
            [pltpu.VMEM((EPW,), jnp.int32)]
            + [pltpu.VMEM((GCH, width), jnp.float32)] * 2
            + [pltpu.SemaphoreType.DMA] * 4
        ),
        mesh=_mesh(),
        compiler_params=_SC_PARAMS,
    )
    def k(tabA, tabB, idxA, idxB, outA, outB,
          idx_v, r0, r1, g0, g1, o0, o1):
        rows = (r0, r1)
        gsem = (g0, g1)
        osem = (o0, o1)
        c = lax.axis_index("c")
        s = lax.axis_index("s")
        w = s * NC + c
        base = w * EPW
        for tab, idx, out in ((tabA, idxA, outA), (tabB, idxB, outB)):
            pltpu.sync_copy(idx.at[w], idx_v)

            def _gat(j, b):
                pltpu.async_copy(
                    tab.at[idx_v.at[pl.ds(j * GCH, GCH)]], rows[b], gsem[b])

            def _gat_wait(j, b):
                pltpu.make_async_copy(
                    tab.at[idx_v.at[pl.ds(j * GCH, GCH)]], rows[b],
                    gsem[b]).wait()

            def _out(j, b):
                pltpu.async_copy(
                    rows[b], out.at[pl.ds(base + j * GCH, GCH)], osem[b])

            def _out_wait(j, b):
                pltpu.make_async_copy(
                    rows[b], out.at[pl.ds(base + j * GCH, GCH)],
                    osem[b]).wait()

            _gat(0, 0)
            _gat(1, 1)

            def grp(g, _):
                for b in (0, 1):
                    j = 2 * g + b
                    _gat_wait(j, b)
                    _out(j, b)
                    _out_wait(j, b)
                    _gat(j + 2, b)
                return 0

            lax.fori_loop(0, (GJ - 3) // 2, grp, 0, unroll=False)
            _gat_wait(GJ - 3, 0)
            _out(GJ - 3, 0)
            _out_wait(GJ - 3, 0)
            _gat(GJ - 1, 0)
            _gat_wait(GJ - 2, 1)
            _out(GJ - 2, 1)
            _gat_wait(GJ - 1, 0)
            _out(GJ - 1, 0)
            _out_wait(GJ - 2, 1)
            _out_wait(GJ - 1, 0)

    return k


@functools.lru_cache(maxsize=None)
def _scatter_kernel():
    @functools.partial(
        pl.kernel,
        out_type=jax.ShapeDtypeStruct((NC, N, SW), jnp.float32),
        scratch_types=(
            [pltpu.VMEM((1, SCH), jnp.int32)] * 2
            + [pltpu.VMEM((SCH, SW), jnp.float32)] * 2
            + [pltpu.VMEM_SHARED((N, SW), jnp.float32)]
            + [pltpu.SemaphoreType.DMA] * 6
        ),
        mesh=_mesh(),
        compiler_params=_SC_PARAMS,
    )
    def _scatter_add(vals, zeros, idx_m, out,
                     i0, i1, r0, r1, acc,
                     l0, l1, s0, s1, q0, q1):
        rows = (r0, r1)
        idxb = (i0, i1)
        lsem = (l0, l1)
        ssem = (s0, s1)
        isem = (q0, q1)
        c = lax.axis_index("c")
        s = lax.axis_index("s")
        base = s * SOPS * SCH
        pltpu.sync_copy(zeros.at[pl.ds(s * ZST, ZST)],
                        acc.at[pl.ds(s * ZST, ZST)])
        plsc.subcore_barrier()

        def _load(j, b):
            pltpu.async_copy(vals.at[c, pl.ds(base + j * SCH, SCH)],
                             rows[b], lsem[b])
            pltpu.async_copy(idx_m.at[pl.ds(s * SOPS + j, 1)],
                             idxb[b], isem[b])

        def _load_wait(j, b):
            pltpu.make_async_copy(vals.at[c, pl.ds(base + j * SCH, SCH)],
                                  rows[b], lsem[b]).wait()
            pltpu.make_async_copy(idx_m.at[pl.ds(s * SOPS + j, 1)],
                                  idxb[b], isem[b]).wait()

        def _add(j, b):
            pltpu.async_copy(rows[b], acc.at[idxb[b].at[0]],
                             ssem[b], add=True)

        def _add_wait(j, b):
            pltpu.make_async_copy(rows[b], acc.at[idxb[b].at[0]],
                                  ssem[b]).wait()

        _load(0, 0)
        _load(1, 1)

        def grp(g, _):
            for b in (0, 1):
                j = 2 * g + b
                _load_wait(j, b)
                _add(j, b)
                _add_wait(j, b)
                _load(j + 2, b)
            return 0

        lax.fori_loop(0, (SOPS - 2) // 2, grp, 0, unroll=False)
        for b in (0, 1):
            j = SOPS - 2 + b
            _load_wait(j, b)
            _add(j, b)
            _add_wait(j, b)

        @pl.when(s < SXTR)
        def _():
            g = NS * SOPS + s
            pltpu.sync_copy(idx_m.at[pl.ds(g, 1)], i0)
            pltpu.sync_copy(vals.at[c, pl.ds(g * SCH, SCH)], r0)
            pltpu.sync_copy(r0, acc.at[i0.at[0]], add=True)

        plsc.subcore_barrier()
        pltpu.sync_copy(acc.at[pl.ds(s * ZST, ZST)],
                        out.at[c, pl.ds(s * ZST, ZST)])

    return _scatter_add


def _lnmask(z, g, be):
    mask = lax.broadcasted_iota(jnp.int32, z.shape, 1) < HID
    m = jnp.sum(z, axis=1, keepdims=True) * (1.0 / HID)
    d = jnp.where(mask, z - m, 0.0)
    v = jnp.sum(d * d, axis=1, keepdims=True) * (1.0 / HID)
    return d * lax.rsqrt(v + 1e-5) * g + be


def _dot(a, b):
    return jnp.dot(a, b, preferred_element_type=jnp.float32)


def _enc_node_body(x_ref, w1, b1, w2, b2, g, be, pa, pb, h_ref, p_ref, q_ref):
    u = jnp.maximum(_dot(x_ref[...], w1[...]) + b1[...], 0.0)
    z = _dot(u, w2[...]) + b2[...]
    h = _lnmask(z, g[...], be[...])[:, :HID]
    h_ref[...] = h
    p_ref[...] = _dot(h, pa[...])
    q_ref[...] = _dot(h, pb[...])


def _split2(x):
    return jnp.stack([x[:, :SW], x[:, SW:]], axis=0)


def _enc_edge_body(ps_ref, pd_ref, w1, b1, w2, b2, g, be, o_ref):
    d3 = pd_ref[...] - ps_ref[...]
    n2 = jnp.sum(d3 * d3, axis=1, keepdims=True)
    xn = jnp.sqrt(n2)
    safe = jnp.where(xn == 0.0, 1.0, xn)
    lane = lax.broadcasted_iota(jnp.int32, d3.shape, 1)
    attr = jnp.where(lane < 3, d3 / safe, 0.0) + jnp.where(lane == 3, xn, 0.0)
    u = jnp.maximum(_dot(attr, w1[...]) + b1[...], 0.0)
    z = _dot(u, w2[...]) + b2[...]
    o_ref[...] = _split2(_lnmask(z, g[...], be[...])[:, :HID])


def _edge_body(e_ref, gs_ref, gd_ref, w1c, b1, w2, b2, g, be, o_ref):
    e2 = e_ref[...]
    e = jnp.concatenate([e2[0], e2[1]], axis=1)
    u = gs_ref[...] + gd_ref[...] + _dot(e, w1c[...]) + b1[...]
    a = jnp.maximum(u, 0.0)
    z = _dot(a, w2[...]) + b2[...]
    y = _lnmask(z, g[...], be[...])
    o_ref[...] = _split2(e + y[:, :HID])


def _node_body(h_ref, agg_ref, w1a, w1bl, w1bh, b1, w2, b2, g, be, pa, pb,
               ho_ref, p_ref, q_ref):
    h = h_ref[...]
    a2 = agg_ref[...]
    u = jnp.maximum(_dot(h, w1a[...]) + _dot(a2[0], w1bl[...])
                    + _dot(a2[1], w1bh[...]) + b1[...], 0.0)
    z = _dot(u, w2[...]) + b2[...]
    y = _lnmask(z, g[...], be[...])
    hn = h + y[:, :HID]
    ho_ref[...] = hn
    p_ref[...] = _dot(hn, pa[...])
    q_ref[...] = _dot(hn, pb[...])


def _dec_body(h_ref, w1, b1, w2, b2, o_ref):
    u = jnp.maximum(_dot(h_ref[...], w1[...]) + b1[...], 0.0)
    o_ref[...] = _dot(u, w2[...]) + b2[...]


def _row_spec(bs, ncols):
    return pl.BlockSpec((bs, ncols), lambda i: (i, 0))


def _w_spec(shape):
    return pl.BlockSpec(shape, lambda i: (0,) * len(shape))


def _split_spec(bs, sw):
    return pl.BlockSpec((NC, bs, sw), lambda i: (0, i, 0))


def _tc_call(body, nrows, bs, row_ins, w_ins, out_cols, n_out=1):
    def ispec(a):
        if a.ndim == 3:
            return _split_spec(bs, a.shape[2])
        return _row_spec(bs, a.shape[1])

    in_specs = ([ispec(a) for a in row_ins]
                + [_w_spec(w.shape) for w in w_ins])
    out_cols = out_cols if isinstance(out_cols, (list, tuple)) else [out_cols]
    out_specs = [_split_spec(bs, SW) if c == "split" else _row_spec(bs, c)
                 for c in out_cols]
    out_shape = [jax.ShapeDtypeStruct((NC, nrows, SW), jnp.float32)
                 if c == "split" else
                 jax.ShapeDtypeStruct((nrows, c), jnp.float32)
                 for c in out_cols]
    if n_out == 1:
        out_specs, out_shape = out_specs[0], out_shape[0]
    return pl.pallas_call(
        body,
        grid=(nrows // bs,),
        in_specs=in_specs,
        out_specs=out_specs,
        out_shape=out_shape,
    )(*row_ins, *w_ins)


def _pad2(w, rows, cols):
    return jnp.pad(w, ((0, rows - w.shape[0]), (0, cols - w.shape[1])))


def _padr(v, cols):
    return jnp.pad(v.reshape(1, -1), ((0, 0), (0, cols - v.shape[0])))


BE = 5000
BN = 5000


def kernel(x, edge_index, pos, enc_n_W1, enc_n_b1, enc_n_W2, enc_n_b2, enc_n_g, enc_n_be, enc_e_W1, enc_e_b1, enc_e_W2, enc_e_b2, enc_e_g, enc_e_be, mp_eW1, mp_eb1, mp_eW2, mp_eb2, mp_eg, mp_ebe, mp_nW1, mp_nb1, mp_nW2, mp_nb2, mp_ng, mp_nbe, dec_W1, dec_b1, dec_W2, dec_b2):
    src = edge_index[0]
    dst = edge_index[1]
    src_g = src.reshape(NW, EPW)
    dst_g = dst.reshape(NW, EPW)
    zeros_n = jnp.zeros((N, SW), jnp.float32)
    sc_m = dst.reshape(SGRP, SCH)

    def _pq_w(i):
        eW1 = mp_eW1[i]
        return _pad2(eW1[:HID], HID, HP), _pad2(eW1[HID:2 * HID], HID, HP)

    pos_pad = jnp.pad(pos, ((0, 0), (0, POSW - pos.shape[1])))
    ps, pd = _gather2(POSW)(pos_pad, pos_pad, src_g, dst_g)
    e = _tc_call(
        _enc_edge_body, E, BE, [ps, pd],
        [_pad2(enc_e_W1, POSW, HP), _padr(enc_e_b1, HP),
         _pad2(enc_e_W2, HP, HP), _padr(enc_e_b2, HP),
         _padr(enc_e_g, HP), _padr(enc_e_be, HP)],
        "split")
    pa0, pb0 = _pq_w(0)
    h, p, q = _tc_call(
        _enc_node_body, N, BN, [x],
        [_pad2(enc_n_W1, 4, HP), _padr(enc_n_b1, HP),
         _pad2(enc_n_W2, HP, HP), _padr(enc_n_b2, HP),
         _padr(enc_n_g, HP), _padr(enc_n_be, HP), pa0, pb0],
        [HID, HP, HP], n_out=3)

    for i in range(MPN):
        w1c = _pad2(mp_eW1[i][2 * HID:], HID, HP)
        gs, gd = _gather2(HP)(p, q, src_g, dst_g)
        e = _tc_call(
            _edge_body, E, BE, [e, gs, gd],
            [w1c, _padr(mp_eb1[i], HP), _pad2(mp_eW2[i], HP, HP),
             _padr(mp_eb2[i], HP), _padr(mp_eg[i], HP), _padr(mp_ebe[i], HP)],
            "split")
        agg = _scatter_kernel()(e, zeros_n, sc_m)
        pa_n, pb_n = _pq_w(min(i + 1, MPN - 1))
        nW1 = mp_nW1[i]
        h, p, q = _tc_call(
            _node_body, N, BN, [h, agg],
            [_pad2(nW1[:HID], HID, HP),
             nW1[HID:HID + SW], nW1[HID + SW:],
             _padr(mp_nb1[i], HP), _pad2(mp_nW2[i], HP, HP),
             _padr(mp_nb2[i], HP), _padr(mp_ng[i], HP), _padr(mp_nbe[i], HP),
             pa_n, pb_n],
            [HID, HP, HP], n_out=3)

    out = _tc_call(
        _dec_body, N, BN, [h],
        [_pad2(dec_W1, HID, HP), _padr(dec_b1, HP),
         _pad2(dec_W2, HP, 1), dec_b2.reshape(1, 1)],
        1)
    return out


# speedup vs baseline: 1.3114x; 1.3114x over previous
import functools

import jax
import jax.numpy as jnp
from jax import lax
from jax.experimental import pallas as pl
from jax.experimental.pallas import tpu as pltpu
from jax.experimental.pallas import tpu_sc as plsc

N = 50000
E = 800000
HID = 40
HP = 48
POSW = 16
MPN = 15

NC = 2
NS = 16
NW = NC * NS
CH = 128
EPW = E // NW
GCH = 1000
GJ = EPW // GCH

HN = N // NC
ACC = HN + 8
SCH = 640
SGRP = E // SCH
SOPS = 78
SXTR = SGRP - NS * SOPS
ZST = ACC // NS
OST_LO = HN // NS


def _mesh():
    return plsc.VectorSubcoreMesh(
        core_axis_name="c", subcore_axis_name="s",
        num_cores=NC, num_subcores=NS)


_SC_PARAMS = pltpu.CompilerParams(use_tc_tiling_on_sc=False)


@functools.lru_cache(maxsize=None)
def _gather2(width):
    @functools.partial(
        pl.kernel,
        out_type=[jax.ShapeDtypeStruct((E, width), jnp.float32)] * 2,
        scratch_types=(
            [pltpu.VMEM((EPW,), jnp.int32)]
            + [pltpu.VMEM((GCH, width), jnp.float32)] * 2
            + [pltpu.SemaphoreType.DMA] * 4
        ),
        mesh=_mesh(),
        compiler_params=_SC_PARAMS,
    )
    def k(tabA, tabB, idxA, idxB, outA, outB,
          idx_v, r0, r1, g0, g1, o0, o1):
        rows = (r0, r1)
        gsem = (g0, g1)
        osem = (o0, o1)
        c = lax.axis_index("c")
        s = lax.axis_index("s")
        w = s * NC + c
        base = w * EPW
        for tab, idx, out in ((tabA, idxA, outA), (tabB, idxB, outB)):
            pltpu.sync_copy(idx.at[w], idx_v)

            def _gat(j, b):
                pltpu.async_copy(
                    tab.at[idx_v.at[pl.ds(j * GCH, GCH)]], rows[b], gsem[b])

            def _gat_wait(j, b):
                pltpu.make_async_copy(
                    tab.at[idx_v.at[pl.ds(j * GCH, GCH)]], rows[b],
                    gsem[b]).wait()

            def _out(j, b):
                pltpu.async_copy(
                    rows[b], out.at[pl.ds(base + j * GCH, GCH)], osem[b])

            def _out_wait(j, b):
                pltpu.make_async_copy(
                    rows[b], out.at[pl.ds(base + j * GCH, GCH)],
                    osem[b]).wait()

            _gat(0, 0)
            _gat(1, 1)

            def grp(g, _):
                for b in (0, 1):
                    j = 2 * g + b
                    _gat_wait(j, b)
                    _out(j, b)
                    _out_wait(j, b)
                    _gat(j + 2, b)
                return 0

            lax.fori_loop(0, (GJ - 3) // 2, grp, 0, unroll=False)
            _gat_wait(GJ - 3, 0)
            _out(GJ - 3, 0)
            _out_wait(GJ - 3, 0)
            _gat(GJ - 1, 0)
            _gat_wait(GJ - 2, 1)
            _out(GJ - 2, 1)
            _gat_wait(GJ - 1, 0)
            _out(GJ - 1, 0)
            _out_wait(GJ - 2, 1)
            _out_wait(GJ - 1, 0)

    return k


@functools.lru_cache(maxsize=None)
def _scatter_kernel():
    @functools.partial(
        pl.kernel,
        out_type=jax.ShapeDtypeStruct((N, HID), jnp.float32),
        scratch_types=(
            [pltpu.VMEM((1, SCH), jnp.int32)] * 2
            + [pltpu.VMEM((SCH, HID), jnp.float32)] * 2
            + [pltpu.VMEM_SHARED((ACC, HID), jnp.float32)]
            + [pltpu.SemaphoreType.DMA] * 6
        ),
        mesh=_mesh(),
        compiler_params=_SC_PARAMS,
    )
    def _scatter_add(vals, zeros, idx_m, out,
                     i0, i1, r0, r1, acc,
                     l0, l1, s0, s1, q0, q1):
        rows = (r0, r1)
        idxb = (i0, i1)
        lsem = (l0, l1)
        ssem = (s0, s1)
        isem = (q0, q1)
        c = lax.axis_index("c")
        s = lax.axis_index("s")
        base = s * SOPS * SCH
        pltpu.sync_copy(zeros.at[pl.ds(s * ZST, ZST)],
                        acc.at[pl.ds(s * ZST, ZST)])
        plsc.subcore_barrier()

        def _load(j, b):
            pltpu.async_copy(vals.at[pl.ds(base + j * SCH, SCH)],
                             rows[b], lsem[b])
            pltpu.async_copy(idx_m.at[c, pl.ds(s * SOPS + j, 1)],
                             idxb[b], isem[b])

        def _load_wait(j, b):
            pltpu.make_async_copy(vals.at[pl.ds(base + j * SCH, SCH)],
                                  rows[b], lsem[b]).wait()
            pltpu.make_async_copy(idx_m.at[c, pl.ds(s * SOPS + j, 1)],
                                  idxb[b], isem[b]).wait()

        def _add(j, b):
            pltpu.async_copy(rows[b], acc.at[idxb[b].at[0]],
                             ssem[b], add=True)

        def _add_wait(j, b):
            pltpu.make_async_copy(rows[b], acc.at[idxb[b].at[0]],
                                  ssem[b]).wait()

        _load(0, 0)
        _load(1, 1)

        def grp(g, _):
            for b in (0, 1):
                j = 2 * g + b
                _load_wait(j, b)
                _add(j, b)
                _add_wait(j, b)
                _load(j + 2, b)
            return 0

        lax.fori_loop(0, (SOPS - 2) // 2, grp, 0, unroll=False)
        for b in (0, 1):
            j = SOPS - 2 + b
            _load_wait(j, b)
            _add(j, b)
            _add_wait(j, b)

        @pl.when(s < SXTR)
        def _():
            g = NS * SOPS + s
            pltpu.sync_copy(idx_m.at[c, pl.ds(g, 1)], i0)
            pltpu.sync_copy(vals.at[pl.ds(g * SCH, SCH)], r0)
            pltpu.sync_copy(r0, acc.at[i0.at[0]], add=True)

        plsc.subcore_barrier()
        ostart = s * OST_LO + jnp.minimum(s, 8)

        @pl.when(s < 8)
        def _():
            pltpu.sync_copy(acc.at[pl.ds(ostart, OST_LO + 1)],
                            out.at[pl.ds(c * HN + ostart, OST_LO + 1)])

        @pl.when(s >= 8)
        def _():
            pltpu.sync_copy(acc.at[pl.ds(ostart, OST_LO)],
                            out.at[pl.ds(c * HN + ostart, OST_LO)])

    return _scatter_add


def _ln(z, g, be):
    m = jnp.sum(z, axis=1, keepdims=True) * (1.0 / HID)
    d = z - m
    v = jnp.sum(d * d, axis=1, keepdims=True) * (1.0 / HID)
    return d * lax.rsqrt(v + 1e-5) * g + be


def _dot(a, b):
    return jnp.dot(a, b, preferred_element_type=jnp.float32)


def _enc_node_body(x_ref, w1, b1, w2, b2, g, be, pa, pb, h_ref, p_ref, q_ref):
    u = jnp.maximum(_dot(x_ref[...], w1[...]) + b1[...], 0.0)
    z = _dot(u, w2[...]) + b2[...]
    h = _ln(z, g[...], be[...])
    h_ref[...] = h
    p_ref[...] = _dot(h, pa[...])
    q_ref[...] = _dot(h, pb[...])


def _enc_edge_body(ps_ref, pd_ref, w1, b1, w2, b2, g, be, o_ref):
    d3 = pd_ref[...] - ps_ref[...]
    n2 = jnp.sum(d3 * d3, axis=1, keepdims=True)
    xn = jnp.sqrt(n2)
    safe = jnp.where(xn == 0.0, 1.0, xn)
    lane = lax.broadcasted_iota(jnp.int32, d3.shape, 1)
    attr = jnp.where(lane < 3, d3 / safe, 0.0) + jnp.where(lane == 3, xn, 0.0)
    u = jnp.maximum(_dot(attr, w1[...]) + b1[...], 0.0)
    z = _dot(u, w2[...]) + b2[...]
    o_ref[...] = _ln(z, g[...], be[...])


def _edge_body(e_ref, gs_ref, gd_ref, w1c, b1, w2, b2, g, be, o_ref):
    e = e_ref[...]
    u = (gs_ref[:, :HID] + gd_ref[:, :HID]
         + _dot(e, w1c[...]) + b1[...])
    a = jnp.maximum(u, 0.0)
    z = _dot(a, w2[...]) + b2[...]
    o_ref[...] = e + _ln(z, g[...], be[...])


def _node_body(h_ref, agg_ref, w1a, w1b, b1, w2, b2, g, be, pa, pb,
               ho_ref, p_ref, q_ref):
    h = h_ref[...]
    agg = agg_ref[...]
    u = jnp.maximum(_dot(h, w1a[...]) + _dot(agg, w1b[...]) + b1[...], 0.0)
    z = _dot(u, w2[...]) + b2[...]
    hn = h + _ln(z, g[...], be[...])
    ho_ref[...] = hn
    p_ref[...] = _dot(hn, pa[...])
    q_ref[...] = _dot(hn, pb[...])


def _dec_body(h_ref, w1, b1, w2, b2, o_ref):
    u = jnp.maximum(_dot(h_ref[...], w1[...]) + b1[...], 0.0)
    o_ref[...] = _dot(u, w2[...]) + b2[...]


def _row_spec(bs, ncols):
    return pl.BlockSpec((bs, ncols), lambda i: (i, 0))


def _w_spec(shape):
    return pl.BlockSpec(shape, lambda i: (0,) * len(shape))


def _tc_call(body, nrows, bs, row_ins, w_ins, out_cols, n_out=1):
    in_specs = ([_row_spec(bs, a.shape[1]) for a in row_ins]
                + [_w_spec(w.shape) for w in w_ins])
    out_cols = out_cols if isinstance(out_cols, (list, tuple)) else [out_cols]
    out_specs = [_row_spec(bs, c) for c in out_cols]
    out_shape = [jax.ShapeDtypeStruct((nrows, c), jnp.float32) for c in out_cols]
    if n_out == 1:
        out_specs, out_shape = out_specs[0], out_shape[0]
    return pl.pallas_call(
        body,
        grid=(nrows // bs,),
        in_specs=in_specs,
        out_specs=out_specs,
        out_shape=out_shape,
    )(*row_ins, *w_ins)


def _pad2(w, rows, cols):
    return jnp.pad(w, ((0, rows - w.shape[0]), (0, cols - w.shape[1])))


def _padr(v, cols):
    return jnp.pad(v.reshape(1, -1), ((0, 0), (0, cols - v.shape[0])))


BE = 5000
BN = 5000


def kernel(x, edge_index, pos, enc_n_W1, enc_n_b1, enc_n_W2, enc_n_b2, enc_n_g, enc_n_be, enc_e_W1, enc_e_b1, enc_e_W2, enc_e_b2, enc_e_g, enc_e_be, mp_eW1, mp_eb1, mp_eW2, mp_eb2, mp_eg, mp_ebe, mp_nW1, mp_nb1, mp_nW2, mp_nb2, mp_ng, mp_nbe, dec_W1, dec_b1, dec_W2, dec_b2):
    src = edge_index[0]
    dst = edge_index[1]
    src_g = src.reshape(NW, EPW)
    dst_g = dst.reshape(NW, EPW)
    zeros_n = jnp.zeros((N, HID), jnp.float32)

    sc_m = jnp.stack([
        jnp.where(dst < HN, dst, HN).reshape(SGRP, SCH),
        jnp.where(dst >= HN, dst - HN, HN).reshape(SGRP, SCH)])

    def _pq_w(i):
        eW1 = mp_eW1[i]
        return _pad2(eW1[:HID], HID, HP), _pad2(eW1[HID:2 * HID], HID, HP)

    pos_pad = jnp.pad(pos, ((0, 0), (0, POSW - pos.shape[1])))
    ps, pd = _gather2(POSW)(pos_pad, pos_pad, src_g, dst_g)
    e = _tc_call(
        _enc_edge_body, E, BE, [ps, pd],
        [_pad2(enc_e_W1, POSW, HID), enc_e_b1.reshape(1, -1),
         enc_e_W2, enc_e_b2.reshape(1, -1),
         enc_e_g.reshape(1, -1), enc_e_be.reshape(1, -1)],
        HID)
    pa0, pb0 = _pq_w(0)
    h, p, q = _tc_call(
        _enc_node_body, N, BN, [x],
        [enc_n_W1, enc_n_b1.reshape(1, -1),
         enc_n_W2, enc_n_b2.reshape(1, -1),
         enc_n_g.reshape(1, -1), enc_n_be.reshape(1, -1), pa0, pb0],
        [HID, HP, HP], n_out=3)

    for i in range(MPN):
        w1c = mp_eW1[i][2 * HID:]
        gs, gd = _gather2(HP)(p, q, src_g, dst_g)
        e = _tc_call(
            _edge_body, E, BE, [e, gs, gd],
            [w1c, mp_eb1[i].reshape(1, -1), mp_eW2[i],
             mp_eb2[i].reshape(1, -1), mp_eg[i].reshape(1, -1),
             mp_ebe[i].reshape(1, -1)],
            HID)
        agg = _scatter_kernel()(e, zeros_n, sc_m)
        pa_n, pb_n = _pq_w(min(i + 1, MPN - 1))
        nW1 = mp_nW1[i]
        h, p, q = _tc_call(
            _node_body, N, BN, [h, agg],
            [nW1[:HID], nW1[HID:],
             mp_nb1[i].reshape(1, -1), mp_nW2[i],
             mp_nb2[i].reshape(1, -1), mp_ng[i].reshape(1, -1),
             mp_nbe[i].reshape(1, -1), pa_n, pb_n],
            [HID, HP, HP], n_out=3)

    out = _tc_call(
        _dec_body, N, BN, [h],
        [dec_W1, dec_b1.reshape(1, -1), dec_W2, dec_b2.reshape(1, 1)],
        1)
    return out
